# baseline (speedup 1.0000x reference)
import functools

import jax
import jax.numpy as jnp
from jax import lax
from jax.experimental import pallas as pl
from jax.experimental.pallas import tpu as pltpu
from jax.experimental.pallas import tpu_sc as plsc

VOCAB = 1000000
EMBED = 32
B, T = 4096, 200
N = B * T

_info = plsc.get_sparse_core_info()
NC, NS = _info.num_cores, _info.num_subcores
NW = NC * NS
BLK = 128
PER_W = N // NW
GR = 4

_mesh = plsc.VectorSubcoreMesh(core_axis_name="c", subcore_axis_name="s")


@functools.partial(
    pl.kernel,
    mesh=_mesh,
    out_type=jax.ShapeDtypeStruct((T, EMBED // 8, B // BLK, 8, BLK), jnp.float32),
    compiler_params=pltpu.CompilerParams(
        use_tc_tiling_on_sc=False, needs_layout_passes=False
    ),
    scratch_types=[
        pltpu.VMEM((PER_W,), jnp.int32),
        pltpu.VMEM((GR, BLK), jnp.int32),
        pltpu.VMEM((GR, BLK, EMBED), jnp.float32),
        pltpu.VMEM((2, EMBED, BLK), jnp.float32),
        pltpu.SemaphoreType.DMA,
        pltpu.SemaphoreType.DMA,
        pltpu.SemaphoreType.DMA,
        pltpu.SemaphoreType.DMA,
        pltpu.SemaphoreType.DMA,
        pltpu.SemaphoreType.DMA,
    ],
)
def _embed_sc(idx_hbm, tab_hbm, out_hbm, idx_all, lst_v, g_v, gt_v,
              sg0, sg1, sg2, sg3, ss0, ss1):
    u = lax.axis_index("s") * NC + lax.axis_index("c")
    sem_g = (sg0, sg1, sg2, sg3)
    sem_s = (ss0, ss1)

    pltpu.sync_copy(idx_hbm.at[pl.ds(u * PER_W, PER_W)], idx_all)

    lane = lax.iota(jnp.int32, 16) * T

    def build_and_fire(t, m):
        for k in range(BLK // 16):
            pos = lane + (k * 16 * T + t)
            lst_v[m, pl.ds(k * 16, 16)] = plsc.load_gather(idx_all, [pos])
        pltpu.async_copy(tab_hbm.at[lst_v.at[m]], g_v.at[m], sem_g[m])

    def wait_gather(m):
        pltpu.make_async_copy(
            tab_hbm.at[pl.ds(0, BLK)], g_v.at[m], sem_g[m]
        ).wait()

    def transpose(m, d):
        def jbody(j, carry):
            cj = jnp.full((16,), j, jnp.int32)
            for k in range(BLK // 16):
                rows = lax.iota(jnp.int32, 16) + (k * 16)
                gt_v[d, j, pl.ds(k * 16, 16)] = plsc.load_gather(
                    g_v.at[m], [rows, cj]
                )
            return carry
        lax.fori_loop(0, EMBED, jbody, 0, unroll=2)

    def fire_stores(t, d):
        for s in range(EMBED // 8):
            pltpu.async_copy(
                gt_v.at[d, pl.ds(8 * s, 8)], out_hbm.at[t, s, u], sem_s[d]
            )

    def wait_stores(d):
        for _ in range(EMBED // 8):
            pltpu.make_async_copy(
                gt_v.at[d, pl.ds(0, 8)], out_hbm.at[0, 0, 0], sem_s[d]
            ).wait()

    for m in range(GR):
        build_and_fire(m, m)

    def step(t, m, d, first):
        wait_gather(m)
        transpose(m, d)
        if not first:
            wait_stores(d)
        fire_stores(t, d)

        @pl.when(t + GR < T)
        def _():
            build_and_fire(t + GR, m)

    for i in range(GR):
        step(i, i, i % 2, first=(i < 2))

    def round_body(q, carry):
        for i in range(GR):
            step(q * GR + i, i, i % 2, first=False)
        return carry

    lax.fori_loop(1, T // GR, round_body, 0)
    wait_stores(0)
    wait_stores(1)


def kernel(tokens, W_E):
    idx = tokens.reshape(N).astype(jnp.int32)
    out5 = _embed_sc(idx, W_E)
    return out5.transpose(2, 4, 0, 1, 3).reshape(B, T, EMBED)
